# baseline (speedup 1.0000x reference)
import functools

import jax
import jax.numpy as jnp
from jax import lax
from jax.experimental import pallas as pl
from jax.experimental.pallas import tpu as pltpu

H = 2048
I = 1024
E = 16
K = 2
NB = 2
S = 4096
NTOK = NB * S
NPAIR = NTOK * K
TT = 256
NTT = NTOK // TT
FT = 128
NFT = NPAIR // FT
SCHED = NFT + E - 1
CHUNK = 512


def _router_rest(logits, eb_ref, e_ref, g_ref):
    lb = logits + eb_ref[...]
    cols = lax.broadcasted_iota(jnp.int32, (TT, E), 1)
    m1 = jnp.max(lb, axis=1, keepdims=True)
    i1 = jnp.min(jnp.where(lb == m1, cols, E + 1), axis=1, keepdims=True)
    masked = jnp.where(cols == i1, -jnp.inf, lb)
    m2 = jnp.max(masked, axis=1, keepdims=True)
    i2 = jnp.min(jnp.where(masked == m2, cols, E + 1), axis=1, keepdims=True)
    s1 = jax.nn.sigmoid(jnp.sum(jnp.where(cols == i1, logits, 0.0), axis=1,
                                keepdims=True))
    s2 = jax.nn.sigmoid(jnp.sum(jnp.where(cols == i2, logits, 0.0), axis=1,
                                keepdims=True))
    den = jnp.abs(s1) + jnp.abs(s2)
    e_ref[...] = jnp.concatenate([i1, i2], axis=1)
    g_ref[...] = jnp.concatenate([s1 / den, s2 / den], axis=1)


def _router_body(x_ref, rw_ref, rb_ref, eb_ref, e_ref, g_ref):
    x = x_ref[...]
    logits = lax.dot_general(x, rw_ref[...], (((1,), (1,)), ((), ())),
                             precision=lax.Precision.HIGHEST,
                             preferred_element_type=jnp.float32) + rb_ref[...]
    _router_rest(logits, eb_ref, e_ref, g_ref)


def _sort_body(e_ref, dest_ref, offs_ref):
    nc = NPAIR // TT
    tri = (lax.broadcasted_iota(jnp.int32, (TT, TT), 0) >=
           lax.broadcasted_iota(jnp.int32, (TT, TT), 1)).astype(jnp.float32)
    tri16 = (lax.broadcasted_iota(jnp.int32, (E, E), 0) >
             lax.broadcasted_iota(jnp.int32, (E, E), 1)).astype(jnp.float32)
    cols3 = lax.broadcasted_iota(jnp.int32, (1, TT, E), 2)

    def hist_step(c, acc):
        ec = e_ref[pl.ds(c, 1), :]
        mc = (ec[:, :, None] == cols3).astype(jnp.float32)[0]
        return acc + jnp.sum(mc, axis=0, keepdims=True)

    hist = lax.fori_loop(0, nc, hist_step, jnp.zeros((1, E), jnp.float32))
    offs = lax.dot_general(hist, tri16, (((1,), (1,)), ((), ())),
                           precision=lax.Precision.HIGHEST,
                           preferred_element_type=jnp.float32)
    offs_ref[...] = offs.astype(jnp.int32)

    def dest_step(c, carry):
        ec = e_ref[pl.ds(c, 1), :]
        mc = (ec[:, :, None] == cols3).astype(jnp.float32)[0]
        incl = jnp.dot(tri, mc, preferred_element_type=jnp.float32)
        ranks = carry + incl - mc
        destc = jnp.sum((offs + ranks) * mc, axis=1, keepdims=True)
        dest_ref[pl.ds(c * TT, TT), :] = destc.astype(jnp.int32)
        return carry + jnp.sum(mc, axis=0, keepdims=True)

    lax.fori_loop(0, nc, dest_step, jnp.zeros((1, E), jnp.float32))


def _sched_body(offs_ref, se_ref, st_ref, lo_ref, hi_ref, sf_ref):
    def outer(e, carry):
        s, prev_t = carry
        off_e = offs_ref[0, e]
        off_n = jnp.where(e == E - 1, NPAIR, offs_ref[0, jnp.minimum(e + 1, E - 1)])
        n_e = off_n - off_e

        def inner(j, c2):
            s2, prev2 = c2
            t = off_e // FT + j
            se_ref[s2] = e
            st_ref[s2] = t
            lo_ref[s2] = jnp.maximum(off_e, t * FT)
            hi_ref[s2] = jnp.minimum(off_n, (t + 1) * FT)
            sf_ref[s2] = jnp.where(t != prev2, 1, 0)
            return s2 + 1, t

        ntiles = jnp.where(n_e > 0, (off_n - 1) // FT - off_e // FT + 1, 0)
        return lax.fori_loop(0, ntiles, inner, (s, prev_t))

    s, prev_t = lax.fori_loop(0, E, outer,
                              (jnp.int32(0), jnp.int32(-1)))
    last_e = se_ref[s - 1]
    last_t = st_ref[s - 1]

    def pad(j, _):
        se_ref[s + j] = last_e
        st_ref[s + j] = last_t
        lo_ref[s + j] = 0
        hi_ref[s + j] = 0
        sf_ref[s + j] = 0
        return 0

    lax.fori_loop(0, SCHED - s, pad, 0)


def _scatter_body(dest_ref, x_ref, xs_ref, sem):
    def start(r, _):
        d = dest_ref[0, 0, r]
        pltpu.make_async_copy(x_ref.at[pl.ds(r // K, 1), :],
                              xs_ref.at[pl.ds(d, 1), :], sem).start()
        return 0

    lax.fori_loop(0, CHUNK, start, 0)

    def drain(r, _):
        pltpu.make_async_copy(x_ref.at[pl.ds(0, 1), :],
                              xs_ref.at[pl.ds(0, 1), :], sem).wait()
        return 0

    lax.fori_loop(0, CHUNK, drain, 0)


def _gather_body(dest_ref, ys_ref, zs_ref, sem):
    def start(r, _):
        d = dest_ref[0, 0, r]
        pltpu.make_async_copy(ys_ref.at[pl.ds(d, 1), :],
                              zs_ref.at[pl.ds(r, 1), :], sem).start()
        return 0

    lax.fori_loop(0, CHUNK, start, 0)

    def drain(r, _):
        pltpu.make_async_copy(ys_ref.at[pl.ds(0, 1), :],
                              zs_ref.at[pl.ds(0, 1), :], sem).wait()
        return 0

    lax.fori_loop(0, CHUNK, drain, 0)


def _ffn_body(se_ref, st_ref, lo_ref, hi_ref, sf_ref,
              xs_ref, w1_ref, b1_ref, w3_ref, b3_ref, w2_ref, b2_ref, ys_ref):
    i = pl.program_id(0)
    lo = lo_ref[i]
    hi = hi_ref[i]
    first = sf_ref[i]
    t = st_ref[i]
    x = xs_ref[...].astype(jnp.bfloat16)
    h1 = lax.dot_general(x, w1_ref[0], (((1,), (1,)), ((), ())),
                         preferred_element_type=jnp.float32) + b1_ref[0]
    h3 = lax.dot_general(x, w3_ref[0], (((1,), (1,)), ((), ())),
                         preferred_element_type=jnp.float32) + b3_ref[0]
    a = (h1 * jax.nn.sigmoid(h1) * h3).astype(jnp.bfloat16)
    y = lax.dot_general(a, w2_ref[0], (((1,), (1,)), ((), ())),
                        preferred_element_type=jnp.float32) + b2_ref[0]
    rows = t * FT + lax.broadcasted_iota(jnp.int32, (FT, 1), 0)
    y = jnp.where((rows >= lo) & (rows < hi), y, 0.0)

    @pl.when(first == 1)
    def _():
        ys_ref[...] = y

    @pl.when(first == 0)
    def _():
        ys_ref[...] = ys_ref[...] + y


def _combine_body(x_ref, zs_ref, g_ref, w1_ref, b1_ref, w3_ref, b3_ref,
                  w2_ref, b2_ref, out_ref):
    x = x_ref[...]
    xx = x.astype(jnp.bfloat16)
    h1 = lax.dot_general(xx, w1_ref[0], (((1,), (1,)), ((), ())),
                         preferred_element_type=jnp.float32) + b1_ref[0]
    h3 = lax.dot_general(xx, w3_ref[0], (((1,), (1,)), ((), ())),
                         preferred_element_type=jnp.float32) + b3_ref[0]
    a = (h1 * jax.nn.sigmoid(h1) * h3).astype(jnp.bfloat16)
    sh = lax.dot_general(a, w2_ref[0], (((1,), (1,)), ((), ())),
                         preferred_element_type=jnp.float32) + b2_ref[0]
    g = g_ref[0]
    z0 = zs_ref[0, :, 0, :]
    z1 = zs_ref[0, :, 1, :]
    out_ref[...] = x + sh + g[:, 0:1] * z0 + g[:, 1:2] * z1


def _pipeline(x, shared_w1_w, shared_w1_b, shared_w2_w, shared_w2_b, shared_w3_w,
              shared_w3_b, routed_w1_w, routed_w1_b, routed_w2_w, routed_w2_b,
              routed_w3_w, routed_w3_b, router_w, router_b, expert_biases):
    xf = x.reshape(NTOK, H)
    rb = router_b.reshape(1, E)
    eb = expert_biases.reshape(1, E)

    e_pairs, gates = pl.pallas_call(
        _router_body,
        grid=(NTT,),
        in_specs=[
            pl.BlockSpec((TT, H), lambda i: (i, 0)),
            pl.BlockSpec((E, H), lambda i: (0, 0)),
            pl.BlockSpec((1, E), lambda i: (0, 0)),
            pl.BlockSpec((1, E), lambda i: (0, 0)),
        ],
        out_specs=[
            pl.BlockSpec((TT, K), lambda i: (i, 0)),
            pl.BlockSpec((TT, K), lambda i: (i, 0)),
        ],
        out_shape=[
            jax.ShapeDtypeStruct((NTOK, K), jnp.int32),
            jax.ShapeDtypeStruct((NTOK, K), jnp.float32),
        ],
    )(xf, router_w, rb, eb)

    e2d = e_pairs.reshape(NPAIR // TT, TT)
    dest, offs = pl.pallas_call(
        _sort_body,
        in_specs=[pl.BlockSpec((NPAIR // TT, TT), lambda: (0, 0))],
        out_specs=[
            pl.BlockSpec((NPAIR, 1), lambda: (0, 0)),
            pl.BlockSpec((1, E), lambda: (0, 0)),
        ],
        out_shape=[
            jax.ShapeDtypeStruct((NPAIR, 1), jnp.int32),
            jax.ShapeDtypeStruct((1, E), jnp.int32),
        ],
    )(e2d)

    sched = pl.pallas_call(
        _sched_body,
        in_specs=[pl.BlockSpec(memory_space=pltpu.SMEM)],
        out_specs=[pl.BlockSpec(memory_space=pltpu.SMEM)] * 5,
        out_shape=[jax.ShapeDtypeStruct((SCHED,), jnp.int32)] * 5,
    )(offs)
    se, st, slo, shi, sf = sched

    dest3 = dest.reshape(NPAIR // CHUNK, 1, CHUNK)
    xs = pl.pallas_call(
        _scatter_body,
        grid=(NPAIR // CHUNK,),
        in_specs=[
            pl.BlockSpec((1, 1, CHUNK), lambda i: (i, 0, 0),
                         memory_space=pltpu.SMEM),
            pl.BlockSpec((CHUNK // K, H), lambda i: (i, 0)),
        ],
        out_specs=pl.BlockSpec(memory_space=pl.ANY),
        out_shape=jax.ShapeDtypeStruct((NPAIR, H), jnp.float32),
        scratch_shapes=[pltpu.SemaphoreType.DMA],
    )(dest3, xf)

    b1r = routed_w1_b.reshape(E, 1, I)
    b3r = routed_w3_b.reshape(E, 1, I)
    b2r = routed_w2_b.reshape(E, 1, H)
    ys = pl.pallas_call(
        _ffn_body,
        grid_spec=pltpu.PrefetchScalarGridSpec(
            num_scalar_prefetch=5,
            grid=(SCHED,),
            in_specs=[
                pl.BlockSpec((FT, H), lambda i, se, st, lo, hi, sf: (st[i], 0)),
                pl.BlockSpec((1, I, H), lambda i, se, st, lo, hi, sf: (se[i], 0, 0)),
                pl.BlockSpec((1, 1, I), lambda i, se, st, lo, hi, sf: (se[i], 0, 0)),
                pl.BlockSpec((1, I, H), lambda i, se, st, lo, hi, sf: (se[i], 0, 0)),
                pl.BlockSpec((1, 1, I), lambda i, se, st, lo, hi, sf: (se[i], 0, 0)),
                pl.BlockSpec((1, H, I), lambda i, se, st, lo, hi, sf: (se[i], 0, 0)),
                pl.BlockSpec((1, 1, H), lambda i, se, st, lo, hi, sf: (se[i], 0, 0)),
            ],
            out_specs=pl.BlockSpec((FT, H), lambda i, se, st, lo, hi, sf: (st[i], 0)),
        ),
        out_shape=jax.ShapeDtypeStruct((NPAIR, H), jnp.float32),
    )(se, st, slo, shi, sf, xs, routed_w1_w.astype(jnp.bfloat16), b1r,
      routed_w3_w.astype(jnp.bfloat16), b3r,
      routed_w2_w.astype(jnp.bfloat16), b2r)

    zs = pl.pallas_call(
        _gather_body,
        grid=(NPAIR // CHUNK,),
        in_specs=[
            pl.BlockSpec((1, 1, CHUNK), lambda i: (i, 0, 0),
                         memory_space=pltpu.SMEM),
            pl.BlockSpec(memory_space=pl.ANY),
        ],
        out_specs=pl.BlockSpec((CHUNK, H), lambda i: (i, 0)),
        out_shape=jax.ShapeDtypeStruct((NPAIR, H), jnp.float32),
        scratch_shapes=[pltpu.SemaphoreType.DMA],
    )(dest3, ys)

    zs4 = zs.reshape(NTT, TT, K, H)
    g3 = gates.reshape(NTT, TT, K)
    b1s = shared_w1_b.reshape(1, 1, I)
    b3s = shared_w3_b.reshape(1, 1, I)
    b2s = shared_w2_b.reshape(1, 1, H)
    out = pl.pallas_call(
        _combine_body,
        grid=(NTT,),
        in_specs=[
            pl.BlockSpec((TT, H), lambda i: (i, 0)),
            pl.BlockSpec((1, TT, K, H), lambda i: (i, 0, 0, 0)),
            pl.BlockSpec((1, TT, K), lambda i: (i, 0, 0)),
            pl.BlockSpec((1, I, H), lambda i: (0, 0, 0)),
            pl.BlockSpec((1, 1, I), lambda i: (0, 0, 0)),
            pl.BlockSpec((1, I, H), lambda i: (0, 0, 0)),
            pl.BlockSpec((1, 1, I), lambda i: (0, 0, 0)),
            pl.BlockSpec((1, H, I), lambda i: (0, 0, 0)),
            pl.BlockSpec((1, 1, H), lambda i: (0, 0, 0)),
        ],
        out_specs=pl.BlockSpec((TT, H), lambda i: (i, 0)),
        out_shape=jax.ShapeDtypeStruct((NTOK, H), jnp.float32),
    )(xf, zs4, g3, shared_w1_w.astype(jnp.bfloat16), b1s,
      shared_w3_w.astype(jnp.bfloat16), b3s, shared_w2_w.astype(jnp.bfloat16),
      b2s)

    return dict(e_pairs=e_pairs, gates=gates, dest=dest, offs=offs, xs=xs,
                ys=ys, zs=zs, out=out)


def kernel(x, shared_w1_w, shared_w1_b, shared_w2_w, shared_w2_b, shared_w3_w,
           shared_w3_b, routed_w1_w, routed_w1_b, routed_w2_w, routed_w2_b,
           routed_w3_w, routed_w3_b, router_w, router_b, expert_biases):
    r = _pipeline(x, shared_w1_w, shared_w1_b, shared_w2_w, shared_w2_b,
                  shared_w3_w, shared_w3_b, routed_w1_w, routed_w1_b,
                  routed_w2_w, routed_w2_b, routed_w3_w, routed_w3_b,
                  router_w, router_b, expert_biases)
    return r["out"].reshape(NB, S, H)


# speedup vs baseline: 1.5393x; 1.5393x over previous
import functools

import jax
import jax.numpy as jnp
from jax import lax
from jax.experimental import pallas as pl
from jax.experimental.pallas import tpu as pltpu
from jax.experimental.pallas import tpu_sc as plsc

H = 2048
I = 1024
E = 16
K = 2
NB = 2
S = 4096
NTOK = NB * S
NPAIR = NTOK * K
TT = 256
NTT = NTOK // TT
FT = 128
NFT = NPAIR // FT
SCHED = NFT + E - 1
CHUNK = 512


def _router_rest(logits, eb_ref, e_ref, g_ref):
    lb = logits + eb_ref[...]
    cols = lax.broadcasted_iota(jnp.int32, (TT, E), 1)
    m1 = jnp.max(lb, axis=1, keepdims=True)
    i1 = jnp.min(jnp.where(lb == m1, cols, E + 1), axis=1, keepdims=True)
    masked = jnp.where(cols == i1, -jnp.inf, lb)
    m2 = jnp.max(masked, axis=1, keepdims=True)
    i2 = jnp.min(jnp.where(masked == m2, cols, E + 1), axis=1, keepdims=True)
    s1 = jax.nn.sigmoid(jnp.sum(jnp.where(cols == i1, logits, 0.0), axis=1,
                                keepdims=True))
    s2 = jax.nn.sigmoid(jnp.sum(jnp.where(cols == i2, logits, 0.0), axis=1,
                                keepdims=True))
    den = jnp.abs(s1) + jnp.abs(s2)
    e_ref[...] = jnp.concatenate([i1, i2], axis=1)
    g_ref[...] = jnp.concatenate([s1 / den, s2 / den], axis=1)


def _router_body(x_ref, rw_ref, rb_ref, eb_ref, e_ref, g_ref):
    x = x_ref[...]
    logits = lax.dot_general(x, rw_ref[...], (((1,), (1,)), ((), ())),
                             precision=lax.Precision.HIGHEST,
                             preferred_element_type=jnp.float32) + rb_ref[...]
    _router_rest(logits, eb_ref, e_ref, g_ref)


def _sort_body(e_ref, dest_ref, offs_ref):
    nc = NPAIR // TT
    tri = (lax.broadcasted_iota(jnp.int32, (TT, TT), 0) >=
           lax.broadcasted_iota(jnp.int32, (TT, TT), 1)).astype(jnp.float32)
    tri16 = (lax.broadcasted_iota(jnp.int32, (E, E), 0) >
             lax.broadcasted_iota(jnp.int32, (E, E), 1)).astype(jnp.float32)
    cols3 = lax.broadcasted_iota(jnp.int32, (1, TT, E), 2)

    def hist_step(c, acc):
        ec = e_ref[pl.ds(c, 1), :]
        mc = (ec[:, :, None] == cols3).astype(jnp.float32)[0]
        return acc + jnp.sum(mc, axis=0, keepdims=True)

    hist = lax.fori_loop(0, nc, hist_step, jnp.zeros((1, E), jnp.float32))
    offs = lax.dot_general(hist, tri16, (((1,), (1,)), ((), ())),
                           precision=lax.Precision.HIGHEST,
                           preferred_element_type=jnp.float32)
    offs_ref[...] = offs.astype(jnp.int32)

    def dest_step(c, carry):
        ec = e_ref[pl.ds(c, 1), :]
        mc = (ec[:, :, None] == cols3).astype(jnp.float32)[0]
        incl = jnp.dot(tri, mc, preferred_element_type=jnp.float32)
        ranks = carry + incl - mc
        destc = jnp.sum((offs + ranks) * mc, axis=1, keepdims=True)
        dest_ref[pl.ds(c * TT, TT), :] = destc.astype(jnp.int32)
        return carry + jnp.sum(mc, axis=0, keepdims=True)

    lax.fori_loop(0, nc, dest_step, jnp.zeros((1, E), jnp.float32))


def _sched_body(offs_ref, se_ref, st_ref, lo_ref, hi_ref, sf_ref):
    def outer(e, carry):
        s, prev_t = carry
        off_e = offs_ref[0, e]
        off_n = jnp.where(e == E - 1, NPAIR, offs_ref[0, jnp.minimum(e + 1, E - 1)])
        n_e = off_n - off_e

        def inner(j, c2):
            s2, prev2 = c2
            t = off_e // FT + j
            se_ref[s2] = e
            st_ref[s2] = t
            lo_ref[s2] = jnp.maximum(off_e, t * FT)
            hi_ref[s2] = jnp.minimum(off_n, (t + 1) * FT)
            sf_ref[s2] = jnp.where(t != prev2, 1, 0)
            return s2 + 1, t

        ntiles = jnp.where(n_e > 0, (off_n - 1) // FT - off_e // FT + 1, 0)
        return lax.fori_loop(0, ntiles, inner, (s, prev_t))

    s, prev_t = lax.fori_loop(0, E, outer,
                              (jnp.int32(0), jnp.int32(-1)))
    last_e = se_ref[s - 1]
    last_t = st_ref[s - 1]

    def pad(j, _):
        se_ref[s + j] = last_e
        st_ref[s + j] = last_t
        lo_ref[s + j] = 0
        hi_ref[s + j] = 0
        sf_ref[s + j] = 0
        return 0

    lax.fori_loop(0, SCHED - s, pad, 0)


_NC = 2
_NS = 16
_NW = _NC * _NS
_GCH = 16


def _sc_scatter_body(dest_ref, x_ref, xs_ref, idx_v, rows_v, sem):
    wid = lax.axis_index("s") * _NC + lax.axis_index("c")
    base = wid * (NTOK // _NW)

    def chunk(j, carry):
        off = base + j * _GCH
        pltpu.sync_copy(x_ref.at[pl.ds(off, _GCH)], rows_v)
        pltpu.sync_copy(dest_ref.at[pl.ds(off, _GCH)], idx_v)
        pltpu.async_copy(rows_v, xs_ref.at[idx_v], sem).wait()
        pltpu.sync_copy(dest_ref.at[pl.ds(NTOK + off, _GCH)], idx_v)
        pltpu.async_copy(rows_v, xs_ref.at[idx_v], sem).wait()
        return carry

    lax.fori_loop(0, (NTOK // _NW) // _GCH, chunk, 0)


def _sc_gather_body(dest_ref, ys_ref, zs_ref, idx_v, rows_v, sem):
    wid = lax.axis_index("s") * _NC + lax.axis_index("c")
    base = wid * (NPAIR // _NW)

    def chunk(j, carry):
        off = base + j * _GCH
        pltpu.sync_copy(dest_ref.at[pl.ds(off, _GCH)], idx_v)
        pltpu.async_copy(ys_ref.at[idx_v], rows_v, sem).wait()
        pltpu.sync_copy(rows_v, zs_ref.at[pl.ds(off, _GCH)])
        return carry

    lax.fori_loop(0, (NPAIR // _NW) // _GCH, chunk, 0)


@functools.cache
def _sc_kernels():
    mesh = plsc.VectorSubcoreMesh(core_axis_name="c", subcore_axis_name="s",
                                  num_cores=_NC, num_subcores=_NS)
    scratch = [
        pltpu.VMEM((_GCH,), jnp.int32),
        pltpu.VMEM((_GCH, H), jnp.float32),
        pltpu.SemaphoreType.DMA,
    ]
    scatter = pl.kernel(
        _sc_scatter_body,
        out_type=jax.ShapeDtypeStruct((NPAIR, H), jnp.float32),
        mesh=mesh, scratch_types=scratch)
    gather = pl.kernel(
        _sc_gather_body,
        out_type=jax.ShapeDtypeStruct((NPAIR, H), jnp.float32),
        mesh=mesh, scratch_types=scratch)
    return scatter, gather


def _ffn_body(se_ref, st_ref, lo_ref, hi_ref, sf_ref,
              xs_ref, w1_ref, b1_ref, w3_ref, b3_ref, w2_ref, b2_ref, ys_ref):
    i = pl.program_id(0)
    lo = lo_ref[i]
    hi = hi_ref[i]
    first = sf_ref[i]
    t = st_ref[i]
    x = xs_ref[...]
    h1 = lax.dot_general(x, w1_ref[0], (((1,), (1,)), ((), ())),
                         preferred_element_type=jnp.float32) + b1_ref[0]
    h3 = lax.dot_general(x, w3_ref[0], (((1,), (1,)), ((), ())),
                         preferred_element_type=jnp.float32) + b3_ref[0]
    a = h1 * jax.nn.sigmoid(h1) * h3
    y = lax.dot_general(a, w2_ref[0], (((1,), (1,)), ((), ())),
                        preferred_element_type=jnp.float32) + b2_ref[0]
    rows = t * FT + lax.broadcasted_iota(jnp.int32, (FT, 1), 0)
    y = jnp.where((rows >= lo) & (rows < hi), y, 0.0)

    @pl.when(first == 1)
    def _():
        ys_ref[...] = y

    @pl.when(first == 0)
    def _():
        ys_ref[...] = ys_ref[...] + y


def _combine_body(x_ref, zs_ref, g_ref, w1_ref, b1_ref, w3_ref, b3_ref,
                  w2_ref, b2_ref, out_ref):
    x = x_ref[...]
    h1 = lax.dot_general(x, w1_ref[0], (((1,), (1,)), ((), ())),
                         preferred_element_type=jnp.float32) + b1_ref[0]
    h3 = lax.dot_general(x, w3_ref[0], (((1,), (1,)), ((), ())),
                         preferred_element_type=jnp.float32) + b3_ref[0]
    a = h1 * jax.nn.sigmoid(h1) * h3
    sh = lax.dot_general(a, w2_ref[0], (((1,), (1,)), ((), ())),
                         preferred_element_type=jnp.float32) + b2_ref[0]
    g = g_ref[0]
    z0 = zs_ref[0]
    z1 = zs_ref[1]
    out_ref[...] = x + sh + g[:, 0:1] * z0 + g[:, 1:2] * z1


def _pipeline(x, shared_w1_w, shared_w1_b, shared_w2_w, shared_w2_b, shared_w3_w,
              shared_w3_b, routed_w1_w, routed_w1_b, routed_w2_w, routed_w2_b,
              routed_w3_w, routed_w3_b, router_w, router_b, expert_biases):
    xf = x.reshape(NTOK, H)
    rb = router_b.reshape(1, E)
    eb = expert_biases.reshape(1, E)

    e_pairs, gates = pl.pallas_call(
        _router_body,
        grid=(NTT,),
        in_specs=[
            pl.BlockSpec((TT, H), lambda i: (i, 0)),
            pl.BlockSpec((E, H), lambda i: (0, 0)),
            pl.BlockSpec((1, E), lambda i: (0, 0)),
            pl.BlockSpec((1, E), lambda i: (0, 0)),
        ],
        out_specs=[
            pl.BlockSpec((TT, K), lambda i: (i, 0)),
            pl.BlockSpec((TT, K), lambda i: (i, 0)),
        ],
        out_shape=[
            jax.ShapeDtypeStruct((NTOK, K), jnp.int32),
            jax.ShapeDtypeStruct((NTOK, K), jnp.float32),
        ],
    )(xf, router_w, rb, eb)

    e2d = e_pairs.T.reshape(NPAIR // TT, TT)
    dest, offs = pl.pallas_call(
        _sort_body,
        in_specs=[pl.BlockSpec((NPAIR // TT, TT), lambda: (0, 0))],
        out_specs=[
            pl.BlockSpec((NPAIR, 1), lambda: (0, 0)),
            pl.BlockSpec((1, E), lambda: (0, 0)),
        ],
        out_shape=[
            jax.ShapeDtypeStruct((NPAIR, 1), jnp.int32),
            jax.ShapeDtypeStruct((1, E), jnp.int32),
        ],
    )(e2d)

    sched = pl.pallas_call(
        _sched_body,
        in_specs=[pl.BlockSpec(memory_space=pltpu.SMEM)],
        out_specs=[pl.BlockSpec(memory_space=pltpu.SMEM)] * 5,
        out_shape=[jax.ShapeDtypeStruct((SCHED,), jnp.int32)] * 5,
    )(offs)
    se, st, slo, shi, sf = sched

    destf = dest.reshape(NPAIR)
    sc_scatter, sc_gather = _sc_kernels()
    xs = sc_scatter(destf, xf)

    b1r = routed_w1_b.reshape(E, 1, I)
    b3r = routed_w3_b.reshape(E, 1, I)
    b2r = routed_w2_b.reshape(E, 1, H)
    ys = pl.pallas_call(
        _ffn_body,
        grid_spec=pltpu.PrefetchScalarGridSpec(
            num_scalar_prefetch=5,
            grid=(SCHED,),
            in_specs=[
                pl.BlockSpec((FT, H), lambda i, se, st, lo, hi, sf: (st[i], 0)),
                pl.BlockSpec((1, I, H), lambda i, se, st, lo, hi, sf: (se[i], 0, 0)),
                pl.BlockSpec((1, 1, I), lambda i, se, st, lo, hi, sf: (se[i], 0, 0)),
                pl.BlockSpec((1, I, H), lambda i, se, st, lo, hi, sf: (se[i], 0, 0)),
                pl.BlockSpec((1, 1, I), lambda i, se, st, lo, hi, sf: (se[i], 0, 0)),
                pl.BlockSpec((1, H, I), lambda i, se, st, lo, hi, sf: (se[i], 0, 0)),
                pl.BlockSpec((1, 1, H), lambda i, se, st, lo, hi, sf: (se[i], 0, 0)),
            ],
            out_specs=pl.BlockSpec((FT, H), lambda i, se, st, lo, hi, sf: (st[i], 0)),
        ),
        out_shape=jax.ShapeDtypeStruct((NPAIR, H), jnp.float32),
    )(se, st, slo, shi, sf, xs, routed_w1_w, b1r, routed_w3_w, b3r,
      routed_w2_w, b2r)

    zs = sc_gather(destf, ys)

    zs3 = zs.reshape(K, NTOK, H)
    g3 = gates.reshape(NTT, TT, K)
    b1s = shared_w1_b.reshape(1, 1, I)
    b3s = shared_w3_b.reshape(1, 1, I)
    b2s = shared_w2_b.reshape(1, 1, H)
    out = pl.pallas_call(
        _combine_body,
        grid=(NTT,),
        in_specs=[
            pl.BlockSpec((TT, H), lambda i: (i, 0)),
            pl.BlockSpec((K, TT, H), lambda i: (0, i, 0)),
            pl.BlockSpec((1, TT, K), lambda i: (i, 0, 0)),
            pl.BlockSpec((1, I, H), lambda i: (0, 0, 0)),
            pl.BlockSpec((1, 1, I), lambda i: (0, 0, 0)),
            pl.BlockSpec((1, I, H), lambda i: (0, 0, 0)),
            pl.BlockSpec((1, 1, I), lambda i: (0, 0, 0)),
            pl.BlockSpec((1, H, I), lambda i: (0, 0, 0)),
            pl.BlockSpec((1, 1, H), lambda i: (0, 0, 0)),
        ],
        out_specs=pl.BlockSpec((TT, H), lambda i: (i, 0)),
        out_shape=jax.ShapeDtypeStruct((NTOK, H), jnp.float32),
    )(xf, zs3, g3, shared_w1_w, b1s, shared_w3_w, b3s, shared_w2_w, b2s)

    return dict(e_pairs=e_pairs, gates=gates, dest=dest, offs=offs, xs=xs,
                ys=ys, zs=zs, out=out)


def kernel(x, shared_w1_w, shared_w1_b, shared_w2_w, shared_w2_b, shared_w3_w,
           shared_w3_b, routed_w1_w, routed_w1_b, routed_w2_w, routed_w2_b,
           routed_w3_w, routed_w3_b, router_w, router_b, expert_biases):
    r = _pipeline(x, shared_w1_w, shared_w1_b, shared_w2_w, shared_w2_b,
                  shared_w3_w, shared_w3_b, routed_w1_w, routed_w1_b,
                  routed_w2_w, routed_w2_b, routed_w3_w, routed_w3_b,
                  router_w, router_b, expert_biases)
    return r["out"].reshape(NB, S, H)


# speedup vs baseline: 1.5861x; 1.0304x over previous
import functools

import jax
import jax.numpy as jnp
from jax import lax
from jax.experimental import pallas as pl
from jax.experimental.pallas import tpu as pltpu
from jax.experimental.pallas import tpu_sc as plsc

H = 2048
I = 1024
E = 16
K = 2
NB = 2
S = 4096
NTOK = NB * S
NPAIR = NTOK * K
TT = 256
NTT = NTOK // TT
FT = 128
NFT = NPAIR // FT
SCHED = NFT + E - 1
CHUNK = 512


def _router_rest(logits, eb_ref, e_ref, g_ref):
    lb = logits + eb_ref[...]
    cols = lax.broadcasted_iota(jnp.int32, (TT, E), 1)
    m1 = jnp.max(lb, axis=1, keepdims=True)
    i1 = jnp.min(jnp.where(lb == m1, cols, E + 1), axis=1, keepdims=True)
    masked = jnp.where(cols == i1, -jnp.inf, lb)
    m2 = jnp.max(masked, axis=1, keepdims=True)
    i2 = jnp.min(jnp.where(masked == m2, cols, E + 1), axis=1, keepdims=True)
    s1 = jax.nn.sigmoid(jnp.sum(jnp.where(cols == i1, logits, 0.0), axis=1,
                                keepdims=True))
    s2 = jax.nn.sigmoid(jnp.sum(jnp.where(cols == i2, logits, 0.0), axis=1,
                                keepdims=True))
    den = jnp.abs(s1) + jnp.abs(s2)
    e_ref[...] = jnp.concatenate([i1, i2], axis=1)
    g_ref[...] = jnp.concatenate([s1 / den, s2 / den], axis=1)


def _router_body(x_ref, rw_ref, rb_ref, eb_ref, e_ref, g_ref):
    x = x_ref[...]
    logits = lax.dot_general(x, rw_ref[...], (((1,), (1,)), ((), ())),
                             precision=lax.Precision.HIGHEST,
                             preferred_element_type=jnp.float32) + rb_ref[...]
    _router_rest(logits, eb_ref, e_ref, g_ref)


def _sort_body(e_ref, dest_ref, offs_ref):
    nc = NPAIR // TT
    tri = (lax.broadcasted_iota(jnp.int32, (TT, TT), 0) >=
           lax.broadcasted_iota(jnp.int32, (TT, TT), 1)).astype(jnp.float32)
    tri16 = (lax.broadcasted_iota(jnp.int32, (E, E), 0) >
             lax.broadcasted_iota(jnp.int32, (E, E), 1)).astype(jnp.float32)
    cols3 = lax.broadcasted_iota(jnp.int32, (1, TT, E), 2)

    def hist_step(c, acc):
        ec = e_ref[pl.ds(c, 1), :]
        mc = (ec[:, :, None] == cols3).astype(jnp.float32)[0]
        return acc + jnp.sum(mc, axis=0, keepdims=True)

    hist = lax.fori_loop(0, nc, hist_step, jnp.zeros((1, E), jnp.float32))
    offs = lax.dot_general(hist, tri16, (((1,), (1,)), ((), ())),
                           precision=lax.Precision.HIGHEST,
                           preferred_element_type=jnp.float32)
    offs_ref[...] = offs.astype(jnp.int32)

    def dest_step(c, carry):
        ec = e_ref[pl.ds(c, 1), :]
        mc = (ec[:, :, None] == cols3).astype(jnp.float32)[0]
        incl = jnp.dot(tri, mc, preferred_element_type=jnp.float32)
        ranks = carry + incl - mc
        destc = jnp.sum((offs + ranks) * mc, axis=1, keepdims=True)
        dest_ref[pl.ds(c * TT, TT), :] = destc.astype(jnp.int32)
        return carry + jnp.sum(mc, axis=0, keepdims=True)

    lax.fori_loop(0, nc, dest_step, jnp.zeros((1, E), jnp.float32))


def _sched_body(offs_ref, se_ref, st_ref, lo_ref, hi_ref, sf_ref):
    def outer(e, carry):
        s, prev_t = carry
        off_e = offs_ref[0, e]
        off_n = jnp.where(e == E - 1, NPAIR, offs_ref[0, jnp.minimum(e + 1, E - 1)])
        n_e = off_n - off_e

        def inner(j, c2):
            s2, prev2 = c2
            t = off_e // FT + j
            se_ref[s2] = e
            st_ref[s2] = t
            lo_ref[s2] = jnp.maximum(off_e, t * FT)
            hi_ref[s2] = jnp.minimum(off_n, (t + 1) * FT)
            sf_ref[s2] = jnp.where(t != prev2, 1, 0)
            return s2 + 1, t

        ntiles = jnp.where(n_e > 0, (off_n - 1) // FT - off_e // FT + 1, 0)
        return lax.fori_loop(0, ntiles, inner, (s, prev_t))

    s, prev_t = lax.fori_loop(0, E, outer,
                              (jnp.int32(0), jnp.int32(-1)))
    last_e = se_ref[s - 1]
    last_t = st_ref[s - 1]

    def pad(j, _):
        se_ref[s + j] = last_e
        st_ref[s + j] = last_t
        lo_ref[s + j] = 0
        hi_ref[s + j] = 0
        sf_ref[s + j] = 0
        return 0

    lax.fori_loop(0, SCHED - s, pad, 0)


_NC = 2
_NS = 16
_NW = _NC * _NS
_GCH = 32


def _sc_scatter_body(dest_ref, x_ref, xs_ref, idx_v, rows_v, sem):
    wid = lax.axis_index("s") * _NC + lax.axis_index("c")
    base = wid * (NTOK // _NW)

    def chunk(j, carry):
        off = base + j * _GCH
        pltpu.sync_copy(x_ref.at[pl.ds(off, _GCH)], rows_v)
        pltpu.sync_copy(dest_ref.at[pl.ds(off, _GCH)], idx_v)
        pltpu.async_copy(rows_v, xs_ref.at[idx_v], sem).wait()
        pltpu.sync_copy(dest_ref.at[pl.ds(NTOK + off, _GCH)], idx_v)
        pltpu.async_copy(rows_v, xs_ref.at[idx_v], sem).wait()
        return carry

    lax.fori_loop(0, (NTOK // _NW) // _GCH, chunk, 0)


def _sc_gather_body(dest_ref, ys_ref, zs_ref, idx_v, rows_v, sem):
    wid = lax.axis_index("s") * _NC + lax.axis_index("c")
    base = wid * (NPAIR // _NW)

    def chunk(j, carry):
        off = base + j * _GCH
        pltpu.sync_copy(dest_ref.at[pl.ds(off, _GCH)], idx_v)
        pltpu.async_copy(ys_ref.at[idx_v], rows_v, sem).wait()
        pltpu.sync_copy(rows_v, zs_ref.at[pl.ds(off, _GCH)])
        return carry

    lax.fori_loop(0, (NPAIR // _NW) // _GCH, chunk, 0)


@functools.cache
def _sc_kernels():
    mesh = plsc.VectorSubcoreMesh(core_axis_name="c", subcore_axis_name="s",
                                  num_cores=_NC, num_subcores=_NS)
    scratch = [
        pltpu.VMEM((_GCH,), jnp.int32),
        pltpu.VMEM((_GCH, H), jnp.float32),
        pltpu.SemaphoreType.DMA,
    ]
    scatter = pl.kernel(
        _sc_scatter_body,
        out_type=jax.ShapeDtypeStruct((NPAIR, H), jnp.float32),
        mesh=mesh, scratch_types=scratch)
    gather = pl.kernel(
        _sc_gather_body,
        out_type=jax.ShapeDtypeStruct((NPAIR, H), jnp.float32),
        mesh=mesh, scratch_types=scratch)
    return scatter, gather


def _ffn_body(se_ref, st_ref, lo_ref, hi_ref, sf_ref,
              xs_ref, w1_ref, b1_ref, w3_ref, b3_ref, w2_ref, b2_ref, ys_ref):
    i = pl.program_id(0)
    lo = lo_ref[i]
    hi = hi_ref[i]
    first = sf_ref[i]
    t = st_ref[i]
    x = xs_ref[...]
    h1 = lax.dot_general(x, w1_ref[0], (((1,), (1,)), ((), ())),
                         preferred_element_type=jnp.float32) + b1_ref[0]
    h3 = lax.dot_general(x, w3_ref[0], (((1,), (1,)), ((), ())),
                         preferred_element_type=jnp.float32) + b3_ref[0]
    a = h1 * jax.nn.sigmoid(h1) * h3
    y = lax.dot_general(a, w2_ref[0], (((1,), (1,)), ((), ())),
                        preferred_element_type=jnp.float32) + b2_ref[0]
    rows = t * FT + lax.broadcasted_iota(jnp.int32, (FT, 1), 0)
    y = jnp.where((rows >= lo) & (rows < hi), y, 0.0)

    @pl.when(first == 1)
    def _():
        ys_ref[...] = y

    @pl.when(first == 0)
    def _():
        ys_ref[...] = ys_ref[...] + y


def _combine_body(x_ref, zs_ref, g_ref, w1_ref, b1_ref, w3_ref, b3_ref,
                  w2_ref, b2_ref, out_ref):
    x = x_ref[...]
    h1 = lax.dot_general(x, w1_ref[0], (((1,), (1,)), ((), ())),
                         preferred_element_type=jnp.float32) + b1_ref[0]
    h3 = lax.dot_general(x, w3_ref[0], (((1,), (1,)), ((), ())),
                         preferred_element_type=jnp.float32) + b3_ref[0]
    a = h1 * jax.nn.sigmoid(h1) * h3
    sh = lax.dot_general(a, w2_ref[0], (((1,), (1,)), ((), ())),
                         preferred_element_type=jnp.float32) + b2_ref[0]
    g = g_ref[0]
    z0 = zs_ref[0]
    z1 = zs_ref[1]
    out_ref[...] = x + sh + g[:, 0:1] * z0 + g[:, 1:2] * z1


def _pipeline(x, shared_w1_w, shared_w1_b, shared_w2_w, shared_w2_b, shared_w3_w,
              shared_w3_b, routed_w1_w, routed_w1_b, routed_w2_w, routed_w2_b,
              routed_w3_w, routed_w3_b, router_w, router_b, expert_biases):
    xf = x.reshape(NTOK, H)
    rb = router_b.reshape(1, E)
    eb = expert_biases.reshape(1, E)

    e_pairs, gates = pl.pallas_call(
        _router_body,
        grid=(NTT,),
        in_specs=[
            pl.BlockSpec((TT, H), lambda i: (i, 0)),
            pl.BlockSpec((E, H), lambda i: (0, 0)),
            pl.BlockSpec((1, E), lambda i: (0, 0)),
            pl.BlockSpec((1, E), lambda i: (0, 0)),
        ],
        out_specs=[
            pl.BlockSpec((TT, K), lambda i: (i, 0)),
            pl.BlockSpec((TT, K), lambda i: (i, 0)),
        ],
        out_shape=[
            jax.ShapeDtypeStruct((NTOK, K), jnp.int32),
            jax.ShapeDtypeStruct((NTOK, K), jnp.float32),
        ],
    )(xf, router_w, rb, eb)

    e2d = e_pairs.T.reshape(NPAIR // TT, TT)
    dest, offs = pl.pallas_call(
        _sort_body,
        in_specs=[pl.BlockSpec((NPAIR // TT, TT), lambda: (0, 0))],
        out_specs=[
            pl.BlockSpec((NPAIR, 1), lambda: (0, 0)),
            pl.BlockSpec((1, E), lambda: (0, 0)),
        ],
        out_shape=[
            jax.ShapeDtypeStruct((NPAIR, 1), jnp.int32),
            jax.ShapeDtypeStruct((1, E), jnp.int32),
        ],
    )(e2d)

    sched = pl.pallas_call(
        _sched_body,
        in_specs=[pl.BlockSpec(memory_space=pltpu.SMEM)],
        out_specs=[pl.BlockSpec(memory_space=pltpu.SMEM)] * 5,
        out_shape=[jax.ShapeDtypeStruct((SCHED,), jnp.int32)] * 5,
    )(offs)
    se, st, slo, shi, sf = sched

    destf = dest.reshape(NPAIR)
    sc_scatter, sc_gather = _sc_kernels()
    xs = sc_scatter(destf, xf)

    b1r = routed_w1_b.reshape(E, 1, I)
    b3r = routed_w3_b.reshape(E, 1, I)
    b2r = routed_w2_b.reshape(E, 1, H)
    ys = pl.pallas_call(
        _ffn_body,
        grid_spec=pltpu.PrefetchScalarGridSpec(
            num_scalar_prefetch=5,
            grid=(SCHED,),
            in_specs=[
                pl.BlockSpec((FT, H), lambda i, se, st, lo, hi, sf: (st[i], 0)),
                pl.BlockSpec((1, I, H), lambda i, se, st, lo, hi, sf: (se[i], 0, 0)),
                pl.BlockSpec((1, 1, I), lambda i, se, st, lo, hi, sf: (se[i], 0, 0)),
                pl.BlockSpec((1, I, H), lambda i, se, st, lo, hi, sf: (se[i], 0, 0)),
                pl.BlockSpec((1, 1, I), lambda i, se, st, lo, hi, sf: (se[i], 0, 0)),
                pl.BlockSpec((1, H, I), lambda i, se, st, lo, hi, sf: (se[i], 0, 0)),
                pl.BlockSpec((1, 1, H), lambda i, se, st, lo, hi, sf: (se[i], 0, 0)),
            ],
            out_specs=pl.BlockSpec((FT, H), lambda i, se, st, lo, hi, sf: (st[i], 0)),
        ),
        out_shape=jax.ShapeDtypeStruct((NPAIR, H), jnp.float32),
    )(se, st, slo, shi, sf, xs, routed_w1_w, b1r, routed_w3_w, b3r,
      routed_w2_w, b2r)

    zs = sc_gather(destf, ys)

    zs3 = zs.reshape(K, NTOK, H)
    g3 = gates.reshape(NTT, TT, K)
    b1s = shared_w1_b.reshape(1, 1, I)
    b3s = shared_w3_b.reshape(1, 1, I)
    b2s = shared_w2_b.reshape(1, 1, H)
    out = pl.pallas_call(
        _combine_body,
        grid=(NTT,),
        in_specs=[
            pl.BlockSpec((TT, H), lambda i: (i, 0)),
            pl.BlockSpec((K, TT, H), lambda i: (0, i, 0)),
            pl.BlockSpec((1, TT, K), lambda i: (i, 0, 0)),
            pl.BlockSpec((1, I, H), lambda i: (0, 0, 0)),
            pl.BlockSpec((1, 1, I), lambda i: (0, 0, 0)),
            pl.BlockSpec((1, I, H), lambda i: (0, 0, 0)),
            pl.BlockSpec((1, 1, I), lambda i: (0, 0, 0)),
            pl.BlockSpec((1, H, I), lambda i: (0, 0, 0)),
            pl.BlockSpec((1, 1, H), lambda i: (0, 0, 0)),
        ],
        out_specs=pl.BlockSpec((TT, H), lambda i: (i, 0)),
        out_shape=jax.ShapeDtypeStruct((NTOK, H), jnp.float32),
    )(xf, zs3, g3, shared_w1_w, b1s, shared_w3_w, b3s, shared_w2_w, b2s)

    return dict(e_pairs=e_pairs, gates=gates, dest=dest, offs=offs, xs=xs,
                ys=ys, zs=zs, out=out)


def kernel(x, shared_w1_w, shared_w1_b, shared_w2_w, shared_w2_b, shared_w3_w,
           shared_w3_b, routed_w1_w, routed_w1_b, routed_w2_w, routed_w2_b,
           routed_w3_w, routed_w3_b, router_w, router_b, expert_biases):
    r = _pipeline(x, shared_w1_w, shared_w1_b, shared_w2_w, shared_w2_b,
                  shared_w3_w, shared_w3_b, routed_w1_w, routed_w1_b,
                  routed_w2_w, routed_w2_b, routed_w3_w, routed_w3_b,
                  router_w, router_b, expert_biases)
    return r["out"].reshape(NB, S, H)


# speedup vs baseline: 1.5991x; 1.0082x over previous
import functools

import jax
import jax.numpy as jnp
from jax import lax
from jax.experimental import pallas as pl
from jax.experimental.pallas import tpu as pltpu
from jax.experimental.pallas import tpu_sc as plsc

H = 2048
I = 1024
E = 16
K = 2
NB = 2
S = 4096
NTOK = NB * S
NPAIR = NTOK * K
TT = 256
NTT = NTOK // TT
FT = 128
NFT = NPAIR // FT
SCHED = NFT + E - 1
SORTC = 512


def _router_rest(logits, eb_ref, e_ref, g_ref):
    lb = logits + eb_ref[...]
    cols = lax.broadcasted_iota(jnp.int32, (TT, E), 1)
    m1 = jnp.max(lb, axis=1, keepdims=True)
    i1 = jnp.min(jnp.where(lb == m1, cols, E + 1), axis=1, keepdims=True)
    masked = jnp.where(cols == i1, -jnp.inf, lb)
    m2 = jnp.max(masked, axis=1, keepdims=True)
    i2 = jnp.min(jnp.where(masked == m2, cols, E + 1), axis=1, keepdims=True)
    s1 = jax.nn.sigmoid(jnp.sum(jnp.where(cols == i1, logits, 0.0), axis=1,
                                keepdims=True))
    s2 = jax.nn.sigmoid(jnp.sum(jnp.where(cols == i2, logits, 0.0), axis=1,
                                keepdims=True))
    den = jnp.abs(s1) + jnp.abs(s2)
    e_ref[...] = jnp.concatenate([i1, i2], axis=1)
    g_ref[...] = jnp.concatenate([s1 / den, s2 / den], axis=1)


def _router_body(x_ref, rw_ref, rb_ref, eb_ref, e_ref, g_ref):
    x = x_ref[...]
    logits = lax.dot_general(x, rw_ref[...], (((1,), (1,)), ((), ())),
                             precision=lax.Precision.HIGHEST,
                             preferred_element_type=jnp.float32) + rb_ref[...]
    _router_rest(logits, eb_ref, e_ref, g_ref)


def _sort_body(e_ref, dest_ref, offs_ref):
    nc = NPAIR // SORTC
    tri = (lax.broadcasted_iota(jnp.int32, (SORTC, SORTC), 0) >=
           lax.broadcasted_iota(jnp.int32, (SORTC, SORTC), 1)).astype(jnp.float32)
    tri16 = (lax.broadcasted_iota(jnp.int32, (E, E), 0) >
             lax.broadcasted_iota(jnp.int32, (E, E), 1)).astype(jnp.float32)
    cols3 = lax.broadcasted_iota(jnp.int32, (1, SORTC, E), 2)

    def hist_step(c, acc):
        ec = e_ref[pl.ds(c, 1), :]
        mc = (ec[:, :, None] == cols3).astype(jnp.float32)[0]
        return acc + jnp.sum(mc, axis=0, keepdims=True)

    hist = lax.fori_loop(0, nc, hist_step, jnp.zeros((1, E), jnp.float32))
    offs = lax.dot_general(hist, tri16, (((1,), (1,)), ((), ())),
                           precision=lax.Precision.HIGHEST,
                           preferred_element_type=jnp.float32)
    offs_ref[...] = offs.astype(jnp.int32)

    def dest_step(c, carry):
        ec = e_ref[pl.ds(c, 1), :]
        mc = (ec[:, :, None] == cols3).astype(jnp.float32)[0]
        incl = jnp.dot(tri, mc, preferred_element_type=jnp.float32)
        ranks = carry + incl - mc
        destc = jnp.sum((offs + ranks) * mc, axis=1, keepdims=True)
        dest_ref[pl.ds(c * SORTC, SORTC), :] = destc.astype(jnp.int32)
        return carry + jnp.sum(mc, axis=0, keepdims=True)

    lax.fori_loop(0, nc, dest_step, jnp.zeros((1, E), jnp.float32))


def _sched_body(offs_ref, se_ref, st_ref, lo_ref, hi_ref, sf_ref):
    def outer(e, carry):
        s, prev_t = carry
        off_e = offs_ref[0, e]
        off_n = jnp.where(e == E - 1, NPAIR, offs_ref[0, jnp.minimum(e + 1, E - 1)])
        n_e = off_n - off_e

        def inner(j, c2):
            s2, prev2 = c2
            t = off_e // FT + j
            se_ref[s2] = e
            st_ref[s2] = t
            lo_ref[s2] = jnp.maximum(off_e, t * FT)
            hi_ref[s2] = jnp.minimum(off_n, (t + 1) * FT)
            sf_ref[s2] = jnp.where(t != prev2, 1, 0)
            return s2 + 1, t

        ntiles = jnp.where(n_e > 0, (off_n - 1) // FT - off_e // FT + 1, 0)
        return lax.fori_loop(0, ntiles, inner, (s, prev_t))

    s, prev_t = lax.fori_loop(0, E, outer,
                              (jnp.int32(0), jnp.int32(-1)))
    last_e = se_ref[s - 1]
    last_t = st_ref[s - 1]

    def pad(j, _):
        se_ref[s + j] = last_e
        st_ref[s + j] = last_t
        lo_ref[s + j] = 0
        hi_ref[s + j] = 0
        sf_ref[s + j] = 0
        return 0

    lax.fori_loop(0, SCHED - s, pad, 0)


_NC = 2
_NS = 16
_NW = _NC * _NS
_GCH = 32


def _sc_scatter_body(dest_ref, x_ref, xs_ref, idx_v, rows_v, sem):
    wid = lax.axis_index("s") * _NC + lax.axis_index("c")
    base = wid * (NTOK // _NW)

    def chunk(j, carry):
        off = base + j * _GCH
        pltpu.sync_copy(x_ref.at[pl.ds(off, _GCH)], rows_v)
        pltpu.sync_copy(dest_ref.at[pl.ds(off, _GCH)], idx_v)
        pltpu.async_copy(rows_v, xs_ref.at[idx_v], sem).wait()
        pltpu.sync_copy(dest_ref.at[pl.ds(NTOK + off, _GCH)], idx_v)
        pltpu.async_copy(rows_v, xs_ref.at[idx_v], sem).wait()
        return carry

    lax.fori_loop(0, (NTOK // _NW) // _GCH, chunk, 0)


def _sc_gather_body(dest_ref, ys_ref, zs_ref, idx_v, rows_v, sem):
    wid = lax.axis_index("s") * _NC + lax.axis_index("c")
    base = wid * (NPAIR // _NW)

    def chunk(j, carry):
        off = base + j * _GCH
        pltpu.sync_copy(dest_ref.at[pl.ds(off, _GCH)], idx_v)
        pltpu.async_copy(ys_ref.at[idx_v], rows_v, sem).wait()
        pltpu.sync_copy(rows_v, zs_ref.at[pl.ds(off, _GCH)])
        return carry

    lax.fori_loop(0, (NPAIR // _NW) // _GCH, chunk, 0)


@functools.cache
def _sc_kernels():
    mesh = plsc.VectorSubcoreMesh(core_axis_name="c", subcore_axis_name="s",
                                  num_cores=_NC, num_subcores=_NS)
    scratch = [
        pltpu.VMEM((_GCH,), jnp.int32),
        pltpu.VMEM((_GCH, H), jnp.float32),
        pltpu.SemaphoreType.DMA,
    ]
    scatter = pl.kernel(
        _sc_scatter_body,
        out_type=jax.ShapeDtypeStruct((NPAIR, H), jnp.float32),
        mesh=mesh, scratch_types=scratch)
    gather = pl.kernel(
        _sc_gather_body,
        out_type=jax.ShapeDtypeStruct((NPAIR, H), jnp.float32),
        mesh=mesh, scratch_types=scratch)
    return scatter, gather


def _ffn_body(se_ref, st_ref, lo_ref, hi_ref, sf_ref,
              xs_ref, w1_ref, b1_ref, w3_ref, b3_ref, w2_ref, b2_ref, ys_ref):
    i = pl.program_id(0)
    lo = lo_ref[i]
    hi = hi_ref[i]
    first = sf_ref[i]
    t = st_ref[i]
    x = xs_ref[...]
    h1 = lax.dot_general(x, w1_ref[0], (((1,), (1,)), ((), ())),
                         preferred_element_type=jnp.float32) + b1_ref[0]
    h3 = lax.dot_general(x, w3_ref[0], (((1,), (1,)), ((), ())),
                         preferred_element_type=jnp.float32) + b3_ref[0]
    a = h1 * jax.nn.sigmoid(h1) * h3
    y = lax.dot_general(a, w2_ref[0], (((1,), (1,)), ((), ())),
                        preferred_element_type=jnp.float32) + b2_ref[0]
    rows = t * FT + lax.broadcasted_iota(jnp.int32, (FT, 1), 0)
    y = jnp.where((rows >= lo) & (rows < hi), y, 0.0)

    @pl.when(first == 1)
    def _():
        ys_ref[...] = y

    @pl.when(first == 0)
    def _():
        ys_ref[...] = ys_ref[...] + y


def _combine_body(x_ref, zs_ref, g_ref, w1_ref, b1_ref, w3_ref, b3_ref,
                  w2_ref, b2_ref, out_ref):
    x = x_ref[...]
    h1 = lax.dot_general(x, w1_ref[0], (((1,), (1,)), ((), ())),
                         preferred_element_type=jnp.float32) + b1_ref[0]
    h3 = lax.dot_general(x, w3_ref[0], (((1,), (1,)), ((), ())),
                         preferred_element_type=jnp.float32) + b3_ref[0]
    a = h1 * jax.nn.sigmoid(h1) * h3
    sh = lax.dot_general(a, w2_ref[0], (((1,), (1,)), ((), ())),
                         preferred_element_type=jnp.float32) + b2_ref[0]
    g = g_ref[0]
    z0 = zs_ref[0]
    z1 = zs_ref[1]
    out_ref[...] = x + sh + g[:, 0:1] * z0 + g[:, 1:2] * z1


def _pipeline(x, shared_w1_w, shared_w1_b, shared_w2_w, shared_w2_b, shared_w3_w,
              shared_w3_b, routed_w1_w, routed_w1_b, routed_w2_w, routed_w2_b,
              routed_w3_w, routed_w3_b, router_w, router_b, expert_biases):
    xf = x.reshape(NTOK, H)
    rb = router_b.reshape(1, E)
    eb = expert_biases.reshape(1, E)

    e_pairs, gates = pl.pallas_call(
        _router_body,
        grid=(NTT,),
        in_specs=[
            pl.BlockSpec((TT, H), lambda i: (i, 0)),
            pl.BlockSpec((E, H), lambda i: (0, 0)),
            pl.BlockSpec((1, E), lambda i: (0, 0)),
            pl.BlockSpec((1, E), lambda i: (0, 0)),
        ],
        out_specs=[
            pl.BlockSpec((TT, K), lambda i: (i, 0)),
            pl.BlockSpec((TT, K), lambda i: (i, 0)),
        ],
        out_shape=[
            jax.ShapeDtypeStruct((NTOK, K), jnp.int32),
            jax.ShapeDtypeStruct((NTOK, K), jnp.float32),
        ],
    )(xf, router_w, rb, eb)

    e2d = e_pairs.T.reshape(NPAIR // SORTC, SORTC)
    dest, offs = pl.pallas_call(
        _sort_body,
        in_specs=[pl.BlockSpec((NPAIR // SORTC, SORTC), lambda: (0, 0))],
        out_specs=[
            pl.BlockSpec((NPAIR, 1), lambda: (0, 0)),
            pl.BlockSpec((1, E), lambda: (0, 0)),
        ],
        out_shape=[
            jax.ShapeDtypeStruct((NPAIR, 1), jnp.int32),
            jax.ShapeDtypeStruct((1, E), jnp.int32),
        ],
    )(e2d)

    sched = pl.pallas_call(
        _sched_body,
        in_specs=[pl.BlockSpec(memory_space=pltpu.SMEM)],
        out_specs=[pl.BlockSpec(memory_space=pltpu.SMEM)] * 5,
        out_shape=[jax.ShapeDtypeStruct((SCHED,), jnp.int32)] * 5,
    )(offs)
    se, st, slo, shi, sf = sched

    destf = dest.reshape(NPAIR)
    sc_scatter, sc_gather = _sc_kernels()
    xs = sc_scatter(destf, xf)

    b1r = routed_w1_b.reshape(E, 1, I)
    b3r = routed_w3_b.reshape(E, 1, I)
    b2r = routed_w2_b.reshape(E, 1, H)
    ys = pl.pallas_call(
        _ffn_body,
        grid_spec=pltpu.PrefetchScalarGridSpec(
            num_scalar_prefetch=5,
            grid=(SCHED,),
            in_specs=[
                pl.BlockSpec((FT, H), lambda i, se, st, lo, hi, sf: (st[i], 0)),
                pl.BlockSpec((1, I, H), lambda i, se, st, lo, hi, sf: (se[i], 0, 0)),
                pl.BlockSpec((1, 1, I), lambda i, se, st, lo, hi, sf: (se[i], 0, 0)),
                pl.BlockSpec((1, I, H), lambda i, se, st, lo, hi, sf: (se[i], 0, 0)),
                pl.BlockSpec((1, 1, I), lambda i, se, st, lo, hi, sf: (se[i], 0, 0)),
                pl.BlockSpec((1, H, I), lambda i, se, st, lo, hi, sf: (se[i], 0, 0)),
                pl.BlockSpec((1, 1, H), lambda i, se, st, lo, hi, sf: (se[i], 0, 0)),
            ],
            out_specs=pl.BlockSpec((FT, H), lambda i, se, st, lo, hi, sf: (st[i], 0)),
        ),
        out_shape=jax.ShapeDtypeStruct((NPAIR, H), jnp.float32),
    )(se, st, slo, shi, sf, xs, routed_w1_w, b1r, routed_w3_w, b3r,
      routed_w2_w, b2r)

    zs = sc_gather(destf, ys)

    zs3 = zs.reshape(K, NTOK, H)
    g3 = gates.reshape(NTT, TT, K)
    b1s = shared_w1_b.reshape(1, 1, I)
    b3s = shared_w3_b.reshape(1, 1, I)
    b2s = shared_w2_b.reshape(1, 1, H)
    out = pl.pallas_call(
        _combine_body,
        grid=(NTT,),
        in_specs=[
            pl.BlockSpec((TT, H), lambda i: (i, 0)),
            pl.BlockSpec((K, TT, H), lambda i: (0, i, 0)),
            pl.BlockSpec((1, TT, K), lambda i: (i, 0, 0)),
            pl.BlockSpec((1, I, H), lambda i: (0, 0, 0)),
            pl.BlockSpec((1, 1, I), lambda i: (0, 0, 0)),
            pl.BlockSpec((1, I, H), lambda i: (0, 0, 0)),
            pl.BlockSpec((1, 1, I), lambda i: (0, 0, 0)),
            pl.BlockSpec((1, H, I), lambda i: (0, 0, 0)),
            pl.BlockSpec((1, 1, H), lambda i: (0, 0, 0)),
        ],
        out_specs=pl.BlockSpec((TT, H), lambda i: (i, 0)),
        out_shape=jax.ShapeDtypeStruct((NTOK, H), jnp.float32),
    )(xf, zs3, g3, shared_w1_w, b1s, shared_w3_w, b3s, shared_w2_w, b2s)

    return dict(e_pairs=e_pairs, gates=gates, dest=dest, offs=offs, xs=xs,
                ys=ys, zs=zs, out=out)


def kernel(x, shared_w1_w, shared_w1_b, shared_w2_w, shared_w2_b, shared_w3_w,
           shared_w3_b, routed_w1_w, routed_w1_b, routed_w2_w, routed_w2_b,
           routed_w3_w, routed_w3_b, router_w, router_b, expert_biases):
    r = _pipeline(x, shared_w1_w, shared_w1_b, shared_w2_w, shared_w2_b,
                  shared_w3_w, shared_w3_b, routed_w1_w, routed_w1_b,
                  routed_w2_w, routed_w2_b, routed_w3_w, routed_w3_b,
                  router_w, router_b, expert_biases)
    return r["out"].reshape(NB, S, H)
